# baseline (speedup 1.0000x reference)
import functools

import jax
import jax.numpy as jnp
from jax.experimental import pallas as pl
from jax.experimental.pallas import tpu as pltpu

N, L, D1, D2, C = 200000, 96, 128, 128, 2
BLOCK = 8000
NB = N // BLOCK


def _clam_kernel(h_ref, wfc_ref, bfc_ref, wa_ref, ba_ref, wb_ref, bb_ref,
                 wct_ref, bc_ref, wcls_ref, bcls_ref,
                 araw_ref, logits_ref, yprob_ref, yhat_ref,
                 acc_ref, m_ref, s_ref):
    i = pl.program_id(0)

    @pl.when(i == 0)
    def _init():
        acc_ref[...] = jnp.zeros_like(acc_ref)
        m_ref[0, 0] = -jnp.inf
        s_ref[0, 0] = 0.0

    x = jnp.maximum(
        jnp.dot(h_ref[...], wfc_ref[...], preferred_element_type=jnp.float32)
        + bfc_ref[...], 0.0)
    a = jnp.tanh(
        jnp.dot(x, wa_ref[...], preferred_element_type=jnp.float32)
        + ba_ref[...])
    b = jax.nn.sigmoid(
        jnp.dot(x, wb_ref[...], preferred_element_type=jnp.float32)
        + bb_ref[...])
    A = jnp.sum(a * b * wct_ref[...], axis=1, keepdims=True) + bc_ref[...]
    araw_ref[...] = A

    m_old = m_ref[0, 0]
    m_new = jnp.maximum(m_old, jnp.max(A))
    p = jnp.exp(A - m_new)
    scale = jnp.exp(m_old - m_new)
    s_ref[0, 0] = s_ref[0, 0] * scale + jnp.sum(p)
    pacc = jax.lax.dot_general(p, x, (((0,), (0,)), ((), ())),
                               preferred_element_type=jnp.float32)
    acc_ref[...] = acc_ref[...] * scale + pacc
    m_ref[0, 0] = m_new

    @pl.when(i == NB - 1)
    def _finish():
        M = acc_ref[...] / s_ref[0, 0]
        logits = (jnp.dot(M, wcls_ref[...], preferred_element_type=jnp.float32)
                  + bcls_ref[...])
        logits_ref[...] = logits
        e = jnp.exp(logits - jnp.max(logits))
        yprob_ref[...] = e / jnp.sum(e)
        yhat_ref[...] = jnp.where(logits[:, 1:] > logits[:, :1], 1, 0
                                  ).astype(jnp.int32)


@functools.partial(jax.jit)
def _run(h, W_fc, b_fc, Wa, ba, Wb, bb, Wc, bc, Wcls, bcls):
    full = lambda shape: pl.BlockSpec(shape, lambda i: (0, 0))
    araw, logits, yprob, yhat = pl.pallas_call(
        _clam_kernel,
        grid=(NB,),
        in_specs=[
            pl.BlockSpec((BLOCK, L), lambda i: (i, 0)),
            full((L, D1)),
            full((1, D1)),
            full((D1, D2)),
            full((1, D2)),
            full((D1, D2)),
            full((1, D2)),
            full((1, D2)),
            full((1, 1)),
            full((D1, C)),
            full((1, C)),
        ],
        out_specs=[
            pl.BlockSpec((BLOCK, 1), lambda i: (i, 0)),
            full((1, C)),
            full((1, C)),
            full((1, 1)),
        ],
        out_shape=[
            jax.ShapeDtypeStruct((N, 1), jnp.float32),
            jax.ShapeDtypeStruct((1, C), jnp.float32),
            jax.ShapeDtypeStruct((1, C), jnp.float32),
            jax.ShapeDtypeStruct((1, 1), jnp.int32),
        ],
        scratch_shapes=[
            pltpu.VMEM((1, D1), jnp.float32),
            pltpu.SMEM((1, 1), jnp.float32),
            pltpu.SMEM((1, 1), jnp.float32),
        ],
    )(h, W_fc, b_fc.reshape(1, D1), Wa, ba.reshape(1, D2),
      Wb, bb.reshape(1, D2), Wc.reshape(1, D2), bc.reshape(1, 1),
      Wcls, bcls.reshape(1, C))
    return logits, yprob, yhat, araw.reshape(1, N)


def kernel(h, W_fc, b_fc, Wa, ba, Wb, bb, Wc, bc, Wcls, bcls):
    logits, yprob, yhat, araw = _run(h, W_fc, b_fc, Wa, ba, Wb, bb, Wc, bc,
                                     Wcls, bcls)
    return (logits, yprob, yhat, araw)


# speedup vs baseline: 1.3568x; 1.3568x over previous
import functools

import jax
import jax.numpy as jnp
from jax.experimental import pallas as pl
from jax.experimental.pallas import tpu as pltpu

N, L, D1, D2, C = 200000, 96, 128, 128, 2
BLOCK = 8000
NB = N // BLOCK


def _clam_kernel(h_ref, wfc_ref, bfc_ref, wa_ref, ba_ref, wb_ref, bb_ref,
                 wct_ref, bc_ref, wcls_ref, bcls_ref,
                 araw_ref, logits_ref, yprob_ref, yhat_ref,
                 acc_ref, m_ref, s_ref):
    i = pl.program_id(0)

    @pl.when(i == 0)
    def _init():
        acc_ref[...] = jnp.zeros_like(acc_ref)
        m_ref[0, 0] = -jnp.inf
        s_ref[0, 0] = 0.0

    x = jnp.maximum(
        jnp.dot(h_ref[...], wfc_ref[...], preferred_element_type=jnp.float32)
        + bfc_ref[...], 0.0)
    a = jnp.tanh(
        jnp.dot(x, wa_ref[...], preferred_element_type=jnp.float32)
        + ba_ref[...])
    zb = (jnp.dot(x, wb_ref[...], preferred_element_type=jnp.float32)
          + bb_ref[...])
    b = 0.5 * jnp.tanh(0.5 * zb) + 0.5
    A = jax.lax.dot_general(wct_ref[...], a * b, (((1,), (1,)), ((), ())),
                            preferred_element_type=jnp.float32) + bc_ref[...]
    araw_ref[...] = A.reshape(1, 1, -1)

    m_old = m_ref[0, 0]
    m_new = jnp.maximum(m_old, jnp.max(A))
    p = jnp.exp(A - m_new)
    scale = jnp.exp(m_old - m_new)
    s_ref[0, 0] = s_ref[0, 0] * scale + jnp.sum(p)
    pacc = jnp.dot(p, x, preferred_element_type=jnp.float32)
    acc_ref[...] = acc_ref[...] * scale + pacc
    m_ref[0, 0] = m_new

    @pl.when(i == NB - 1)
    def _finish():
        M = acc_ref[...] / s_ref[0, 0]
        logits = (jnp.dot(M, wcls_ref[...], preferred_element_type=jnp.float32)
                  + bcls_ref[...])
        logits_ref[...] = logits
        e = jnp.exp(logits - jnp.max(logits))
        yprob_ref[...] = e / jnp.sum(e)
        yhat_ref[...] = jnp.where(logits[:, 1:] > logits[:, :1], 1, 0
                                  ).astype(jnp.int32)


@functools.partial(jax.jit)
def _run(h, W_fc, b_fc, Wa, ba, Wb, bb, Wc, bc, Wcls, bcls):
    full = lambda shape: pl.BlockSpec(shape, lambda i: (0, 0))
    araw, logits, yprob, yhat = pl.pallas_call(
        _clam_kernel,
        grid=(NB,),
        in_specs=[
            pl.BlockSpec((BLOCK, L), lambda i: (i, 0)),
            full((L, D1)),
            full((1, D1)),
            full((D1, D2)),
            full((1, D2)),
            full((D1, D2)),
            full((1, D2)),
            full((1, D2)),
            full((1, 1)),
            full((D1, C)),
            full((1, C)),
        ],
        out_specs=[
            pl.BlockSpec((1, 1, BLOCK), lambda i: (i, 0, 0)),
            full((1, C)),
            full((1, C)),
            full((1, 1)),
        ],
        out_shape=[
            jax.ShapeDtypeStruct((NB, 1, BLOCK), jnp.float32),
            jax.ShapeDtypeStruct((1, C), jnp.float32),
            jax.ShapeDtypeStruct((1, C), jnp.float32),
            jax.ShapeDtypeStruct((1, 1), jnp.int32),
        ],
        scratch_shapes=[
            pltpu.VMEM((1, D1), jnp.float32),
            pltpu.SMEM((1, 1), jnp.float32),
            pltpu.SMEM((1, 1), jnp.float32),
        ],
    )(h, W_fc, b_fc.reshape(1, D1), Wa, ba.reshape(1, D2),
      Wb, bb.reshape(1, D2), Wc.reshape(1, D2), bc.reshape(1, 1),
      Wcls, bcls.reshape(1, C))
    return logits, yprob, yhat, araw.reshape(1, N)


def kernel(h, W_fc, b_fc, Wa, ba, Wb, bb, Wc, bc, Wcls, bcls):
    logits, yprob, yhat, araw = _run(h, W_fc, b_fc, Wa, ba, Wb, bb, Wc, bc,
                                     Wcls, bcls)
    return (logits, yprob, yhat, araw)


# speedup vs baseline: 1.4140x; 1.0422x over previous
import functools

import jax
import jax.numpy as jnp
from jax.experimental import pallas as pl
from jax.experimental.pallas import tpu as pltpu

N, L, D1, D2, C = 200000, 96, 128, 128, 2
BLOCK = 20000
NB = N // BLOCK


def _clam_kernel(h_ref, wfc_ref, bfc_ref, wa_ref, ba_ref, wb_ref, bb_ref,
                 wct_ref, bc_ref, wcls_ref, bcls_ref,
                 araw_ref, logits_ref, yprob_ref, yhat_ref,
                 acc_ref, m_ref, s_ref):
    i = pl.program_id(0)

    @pl.when(i == 0)
    def _init():
        acc_ref[...] = jnp.zeros_like(acc_ref)
        m_ref[0, 0] = -jnp.inf
        s_ref[0, 0] = 0.0

    x = jnp.maximum(
        jnp.dot(h_ref[...], wfc_ref[...], preferred_element_type=jnp.float32)
        + bfc_ref[...], 0.0)
    a = jnp.tanh(
        jnp.dot(x, wa_ref[...], preferred_element_type=jnp.float32)
        + ba_ref[...])
    zb = (jnp.dot(x, wb_ref[...], preferred_element_type=jnp.float32)
          + bb_ref[...])
    b = 0.5 * jnp.tanh(0.5 * zb) + 0.5
    A = jax.lax.dot_general(wct_ref[...], a * b, (((1,), (1,)), ((), ())),
                            preferred_element_type=jnp.float32) + bc_ref[...]
    araw_ref[...] = A.reshape(1, 1, -1)

    m_old = m_ref[0, 0]
    m_new = jnp.maximum(m_old, jnp.max(A))
    p = jnp.exp(A - m_new)
    scale = jnp.exp(m_old - m_new)
    s_ref[0, 0] = s_ref[0, 0] * scale + jnp.sum(p)
    pacc = jnp.dot(p, x, preferred_element_type=jnp.float32)
    acc_ref[...] = acc_ref[...] * scale + pacc
    m_ref[0, 0] = m_new

    @pl.when(i == NB - 1)
    def _finish():
        M = acc_ref[...] / s_ref[0, 0]
        logits = (jnp.dot(M, wcls_ref[...], preferred_element_type=jnp.float32)
                  + bcls_ref[...])
        logits_ref[...] = logits
        e = jnp.exp(logits - jnp.max(logits))
        yprob_ref[...] = e / jnp.sum(e)
        yhat_ref[...] = jnp.where(logits[:, 1:] > logits[:, :1], 1, 0
                                  ).astype(jnp.int32)


@functools.partial(jax.jit)
def _run(h, W_fc, b_fc, Wa, ba, Wb, bb, Wc, bc, Wcls, bcls):
    full = lambda shape: pl.BlockSpec(shape, lambda i: (0, 0))
    araw, logits, yprob, yhat = pl.pallas_call(
        _clam_kernel,
        grid=(NB,),
        in_specs=[
            pl.BlockSpec((BLOCK, L), lambda i: (i, 0)),
            full((L, D1)),
            full((1, D1)),
            full((D1, D2)),
            full((1, D2)),
            full((D1, D2)),
            full((1, D2)),
            full((1, D2)),
            full((1, 1)),
            full((D1, C)),
            full((1, C)),
        ],
        out_specs=[
            pl.BlockSpec((1, 1, BLOCK), lambda i: (i, 0, 0)),
            full((1, C)),
            full((1, C)),
            full((1, 1)),
        ],
        out_shape=[
            jax.ShapeDtypeStruct((NB, 1, BLOCK), jnp.float32),
            jax.ShapeDtypeStruct((1, C), jnp.float32),
            jax.ShapeDtypeStruct((1, C), jnp.float32),
            jax.ShapeDtypeStruct((1, 1), jnp.int32),
        ],
        scratch_shapes=[
            pltpu.VMEM((1, D1), jnp.float32),
            pltpu.SMEM((1, 1), jnp.float32),
            pltpu.SMEM((1, 1), jnp.float32),
        ],
    )(h, W_fc, b_fc.reshape(1, D1), Wa, ba.reshape(1, D2),
      Wb, bb.reshape(1, D2), Wc.reshape(1, D2), bc.reshape(1, 1),
      Wcls, bcls.reshape(1, C))
    return logits, yprob, yhat, araw.reshape(1, N)


def kernel(h, W_fc, b_fc, Wa, ba, Wb, bb, Wc, bc, Wcls, bcls):
    logits, yprob, yhat, araw = _run(h, W_fc, b_fc, Wa, ba, Wb, bb, Wc, bc,
                                     Wcls, bcls)
    return (logits, yprob, yhat, araw)
